# Initial kernel scaffold; baseline (speedup 1.0000x reference)
import functools

import jax
import jax.numpy as jnp
from jax import lax
from jax.experimental import pallas as pl
from jax.experimental.pallas import tpu as pltpu
from jax.experimental.pallas import tpu_sc as plsc

_NC = 2
_NS = 16
_NW = _NC * _NS
_FEAT = 16
_BLK = 128
_K = 8


@functools.lru_cache(maxsize=None)
def _make_segment_kernel(n_nodes: int, rows_total: int, chunks: int,
                         acc_rows: int):
    wb_rows = acc_rows // _NS
    rows_per_tile = chunks * _K

    mesh = plsc.VectorSubcoreMesh(
        core_axis_name="c", subcore_axis_name="s",
        num_cores=_NC, num_subcores=_NS)

    @functools.partial(
        pl.kernel,
        out_type=jax.ShapeDtypeStruct((2 * acc_rows, _FEAT), jnp.float32),
        mesh=mesh,
        scratch_types=[
            pltpu.VMEM((_K, _BLK), jnp.int32),
            pltpu.VMEM((_K, _BLK), jnp.int32),
            pltpu.VMEM((_K, _BLK, _FEAT), jnp.float32),
            pltpu.VMEM_SHARED((acc_rows, _FEAT), jnp.float32),
            pltpu.SemaphoreType.DMA,
        ],
    )
    def seg_kernel(h_hbm, src_hbm, dst_hbm, zeros_hbm, out_hbm,
                   src_v, dst_v, rows_v, acc, sem):
        c = lax.axis_index("c")
        s = lax.axis_index("s")
        wid = c * _NS + s

        pltpu.sync_copy(zeros_hbm, acc.at[pl.ds(s * wb_rows, wb_rows)])
        plsc.subcore_barrier()

        base = wid * rows_per_tile

        @pl.loop(0, chunks)
        def _(i):
            r0 = base + i * _K
            pltpu.sync_copy(src_hbm.at[pl.ds(r0, _K)], src_v)
            pltpu.sync_copy(dst_hbm.at[pl.ds(r0, _K)], dst_v)
            gathers = [
                pltpu.async_copy(h_hbm.at[src_v.at[j]], rows_v.at[j], sem)
                for j in range(_K)
            ]
            for j in range(_K):
                gathers[j].wait()
            for j in range(_K):
                pltpu.sync_copy(rows_v.at[j], acc.at[dst_v.at[j]], add=True)

        plsc.subcore_barrier()
        out_base = c * acc_rows + s * wb_rows
        pltpu.sync_copy(acc.at[pl.ds(s * wb_rows, wb_rows)],
                        out_hbm.at[pl.ds(out_base, wb_rows)])

    return seg_kernel


def kernel(x, edge_index, batch):
    n = x.shape[0]
    e = edge_index.shape[1]
    num_it = 3

    edges_per_sweep = _NW * _K * _BLK
    chunks = -(-e // edges_per_sweep)
    e_pad = chunks * edges_per_sweep
    acc_rows = ((n + 1 + _NS - 1) // _NS) * _NS
    wb_rows = acc_rows // _NS

    src = edge_index[0]
    dst = edge_index[1]
    pad = e_pad - e
    src_p = jnp.concatenate(
        [src, jnp.zeros((pad,), jnp.int32)]).reshape(-1, _BLK)
    dst_p = jnp.concatenate(
        [dst, jnp.full((pad,), n, jnp.int32)]).reshape(-1, _BLK)
    zeros = jnp.zeros((wb_rows, _FEAT), jnp.float32)

    h = jnp.concatenate(
        [
            x[:, :8],
            x[:, 9:10],
            x[:, 11:12],
            jnp.zeros((n, 5), jnp.float32),
            jnp.ones((n, 1), jnp.float32),
        ],
        axis=1,
    )

    seg = _make_segment_kernel(n, src_p.shape[0], chunks, acc_rows)

    snaps = [h]
    for _ in range(num_it):
        p = seg(h, src_p, dst_p, zeros)
        agg = p[:acc_rows][:n] + p[acc_rows:][:n]
        deg = jnp.maximum(agg[:, 15:16], 1.0)
        h = 0.5 * (h + agg / deg)
        h = h.at[:, 15].set(1.0)
        snaps.append(h)

    cont = jnp.concatenate([s[:, :8] for s in snaps], axis=1)
    cat1 = jnp.stack([s[:, 8] for s in snaps], axis=1)
    cat2 = jnp.stack([s[:, 9] for s in snaps], axis=1)
    return jnp.concatenate([cont, cat1, cat2], axis=1)


# speedup vs baseline: 113.5107x; 113.5107x over previous
import functools

import jax
import jax.numpy as jnp
from jax import lax
from jax.experimental import pallas as pl
from jax.experimental.pallas import tpu as pltpu
from jax.experimental.pallas import tpu_sc as plsc

_NC = 2
_NS = 16
_NW = _NC * _NS
_FEAT = 16
_BLK = 128
_K = 8


@functools.lru_cache(maxsize=None)
def _make_segment_kernel(n_nodes: int, rows_total: int, chunks: int,
                         acc_rows: int):
    wb_rows = acc_rows // _NS
    rows_per_tile = chunks * _K

    mesh = plsc.VectorSubcoreMesh(
        core_axis_name="c", subcore_axis_name="s",
        num_cores=_NC, num_subcores=_NS)

    @functools.partial(
        pl.kernel,
        out_type=jax.ShapeDtypeStruct((2 * acc_rows, _FEAT), jnp.float32),
        mesh=mesh,
        scratch_types=[
            pltpu.VMEM((_K, _BLK), jnp.int32),
            pltpu.VMEM((_K, _BLK), jnp.int32),
            pltpu.VMEM((_K, _BLK, _FEAT), jnp.float32),
            pltpu.VMEM_SHARED((acc_rows, _FEAT), jnp.float32),
            pltpu.SemaphoreType.DMA,
        ],
        compiler_params=pltpu.CompilerParams(use_tc_tiling_on_sc=False),
    )
    def seg_kernel(h_hbm, src_hbm, dst_hbm, zeros_hbm, out_hbm,
                   src_v, dst_v, rows_v, acc, sem):
        c = lax.axis_index("c")
        s = lax.axis_index("s")
        wid = c * _NS + s

        pltpu.sync_copy(zeros_hbm, acc.at[pl.ds(s * wb_rows, wb_rows)])
        plsc.subcore_barrier()

        base = wid * rows_per_tile

        @pl.loop(0, chunks)
        def _(i):
            r0 = base + i * _K
            pltpu.sync_copy(src_hbm.at[pl.ds(r0, _K)], src_v)
            pltpu.sync_copy(dst_hbm.at[pl.ds(r0, _K)], dst_v)
            gathers = [
                pltpu.async_copy(h_hbm.at[src_v.at[j]], rows_v.at[j], sem)
                for j in range(_K)
            ]
            for j in range(_K):
                gathers[j].wait()
            for j in range(_K):
                pltpu.sync_copy(rows_v.at[j], acc.at[dst_v.at[j]], add=True)

        plsc.subcore_barrier()
        out_base = c * acc_rows + s * wb_rows
        pltpu.sync_copy(acc.at[pl.ds(s * wb_rows, wb_rows)],
                        out_hbm.at[pl.ds(out_base, wb_rows)])

    return seg_kernel


def kernel(x, edge_index, batch):
    n = x.shape[0]
    e = edge_index.shape[1]
    num_it = 3

    edges_per_sweep = _NW * _K * _BLK
    chunks = -(-e // edges_per_sweep)
    e_pad = chunks * edges_per_sweep
    acc_rows = ((n + 1 + 127) // 128) * 128
    wb_rows = acc_rows // _NS

    src = edge_index[0]
    dst = edge_index[1]
    pad = e_pad - e
    src_p = jnp.concatenate(
        [src, jnp.zeros((pad,), jnp.int32)]).reshape(-1, _BLK)
    dst_p = jnp.concatenate(
        [dst, jnp.full((pad,), n, jnp.int32)]).reshape(-1, _BLK)
    zeros = jnp.zeros((wb_rows, _FEAT), jnp.float32)

    h = jnp.concatenate(
        [
            x[:, :8],
            x[:, 9:10],
            x[:, 11:12],
            jnp.zeros((n, 5), jnp.float32),
            jnp.ones((n, 1), jnp.float32),
        ],
        axis=1,
    )

    seg = _make_segment_kernel(n, src_p.shape[0], chunks, acc_rows)

    snaps = [h]
    for _ in range(num_it):
        p = seg(h, src_p, dst_p, zeros)
        agg = p[:acc_rows][:n] + p[acc_rows:][:n]
        deg = jnp.maximum(agg[:, 15:16], 1.0)
        h = 0.5 * (h + agg / deg)
        h = h.at[:, 15].set(1.0)
        snaps.append(h)

    cont = jnp.concatenate([s[:, :8] for s in snaps], axis=1)
    cat1 = jnp.stack([s[:, 8] for s in snaps], axis=1)
    cat2 = jnp.stack([s[:, 9] for s in snaps], axis=1)
    return jnp.concatenate([cont, cat1, cat2], axis=1)


# speedup vs baseline: 168.2211x; 1.4820x over previous
import functools

import jax
import jax.numpy as jnp
from jax import lax
from jax.experimental import pallas as pl
from jax.experimental.pallas import tpu as pltpu
from jax.experimental.pallas import tpu_sc as plsc

_NC = 2
_NS = 16
_NW = _NC * _NS
_FEAT = 16
_BLK = 128
_K = 4


_RING = 3


@functools.lru_cache(maxsize=None)
def _make_segment_kernel(n_nodes: int, rows_total: int, chunks: int,
                         acc_rows: int):
    wb_rows = acc_rows // _NS
    rows_per_tile = chunks * _K
    assert (chunks - 1) % _RING == 0

    mesh = plsc.VectorSubcoreMesh(
        core_axis_name="c", subcore_axis_name="s",
        num_cores=_NC, num_subcores=_NS)

    @functools.partial(
        pl.kernel,
        out_type=jax.ShapeDtypeStruct((2 * acc_rows, _FEAT), jnp.float32),
        mesh=mesh,
        scratch_types=[
            pltpu.VMEM((_RING, _K, _BLK), jnp.int32),
            pltpu.VMEM((_RING, _K, _BLK), jnp.int32),
            pltpu.VMEM((_RING, _K, _BLK, _FEAT), jnp.float32),
            pltpu.VMEM_SHARED((acc_rows, _FEAT), jnp.float32),
            pltpu.SemaphoreType.DMA,
            pltpu.SemaphoreType.DMA,
            pltpu.SemaphoreType.DMA,
        ],
        compiler_params=pltpu.CompilerParams(use_tc_tiling_on_sc=False),
    )
    def seg_kernel(h_hbm, src_hbm, dst_hbm, zeros_hbm, out_hbm,
                   src_v, dst_v, rows_v, acc, sem_i, sem_g, sem_s):
        c = lax.axis_index("c")
        s = lax.axis_index("s")
        wid = c * _NS + s

        pltpu.sync_copy(zeros_hbm, acc.at[pl.ds(s * wb_rows, wb_rows)])
        plsc.subcore_barrier()

        base = wid * rows_per_tile
        max_r0 = rows_total - _K

        def prefetch_idx(chunk, buf):
            r0 = jnp.minimum(base + chunk * _K, max_r0)
            pltpu.async_copy(src_hbm.at[pl.ds(r0, _K)], src_v.at[buf], sem_i)
            pltpu.async_copy(dst_hbm.at[pl.ds(r0, _K)], dst_v.at[buf], sem_i)

        def wait_idx(buf):
            pltpu.make_async_copy(
                src_hbm.at[pl.ds(0, _K)], src_v.at[buf], sem_i).wait()
            pltpu.make_async_copy(
                src_hbm.at[pl.ds(0, _K)], dst_v.at[buf], sem_i).wait()

        def issue_gathers(buf):
            for j in range(_K):
                pltpu.async_copy(h_hbm.at[src_v.at[buf].at[j]],
                                 rows_v.at[buf].at[j], sem_g)

        def drain_rows(buf, sem):
            for j in range(_K):
                pltpu.make_async_copy(h_hbm.at[pl.ds(0, _BLK)],
                                      rows_v.at[buf].at[j], sem).wait()

        def issue_scatters(buf):
            for j in range(_K):
                pltpu.async_copy(rows_v.at[buf].at[j],
                                 acc.at[dst_v.at[buf].at[j]], sem_s, add=True)

        prefetch_idx(0, 0)
        prefetch_idx(1, 1)
        wait_idx(0)
        issue_gathers(0)

        groups = (chunks - 1) // _RING

        @pl.loop(0, groups)
        def _(g):
            for b in range(_RING):
                i = g * _RING + b
                if b < 2:
                    @pl.when(g > 0)
                    def _():
                        drain_rows(b, sem_s)
                else:
                    drain_rows(b, sem_s)
                prefetch_idx(i + 2, (b + 2) % _RING)
                wait_idx((b + 1) % _RING)
                issue_gathers((b + 1) % _RING)
                drain_rows(b, sem_g)
                issue_scatters(b)

        last = chunks - 1
        drain_rows(0, sem_g)
        issue_scatters(0)
        for b in ((last - 2) % _RING, (last - 1) % _RING, 0):
            drain_rows(b, sem_s)
        wait_idx(1)

        plsc.subcore_barrier()
        out_base = c * acc_rows + s * wb_rows
        pltpu.sync_copy(acc.at[pl.ds(s * wb_rows, wb_rows)],
                        out_hbm.at[pl.ds(out_base, wb_rows)])

    return seg_kernel


def kernel(x, edge_index, batch):
    n = x.shape[0]
    e = edge_index.shape[1]
    num_it = 3

    edges_per_sweep = _NW * _K * _BLK
    chunks = -(-e // edges_per_sweep)
    while (chunks - 1) % _RING:
        chunks += 1
    e_pad = chunks * edges_per_sweep
    acc_rows = ((n + 1 + 127) // 128) * 128
    wb_rows = acc_rows // _NS

    src = edge_index[0]
    dst = edge_index[1]
    pad = e_pad - e
    src_p = jnp.concatenate(
        [src, jnp.zeros((pad,), jnp.int32)]).reshape(-1, _BLK)
    dst_p = jnp.concatenate(
        [dst, jnp.full((pad,), n, jnp.int32)]).reshape(-1, _BLK)
    zeros = jnp.zeros((wb_rows, _FEAT), jnp.float32)

    h = jnp.concatenate(
        [
            x[:, :8],
            x[:, 9:10],
            x[:, 11:12],
            jnp.zeros((n, 5), jnp.float32),
            jnp.ones((n, 1), jnp.float32),
        ],
        axis=1,
    )

    seg = _make_segment_kernel(n, src_p.shape[0], chunks, acc_rows)

    snaps = [h]
    for _ in range(num_it):
        p = seg(h, src_p, dst_p, zeros)
        agg = p[:acc_rows][:n] + p[acc_rows:][:n]
        deg = jnp.maximum(agg[:, 15:16], 1.0)
        h = 0.5 * (h + agg / deg)
        h = h.at[:, 15].set(1.0)
        snaps.append(h)

    cont = jnp.concatenate([s[:, :8] for s in snaps], axis=1)
    cat1 = jnp.stack([s[:, 8] for s in snaps], axis=1)
    cat2 = jnp.stack([s[:, 9] for s in snaps], axis=1)
    return jnp.concatenate([cont, cat1, cat2], axis=1)


# speedup vs baseline: 168.9380x; 1.0043x over previous
import functools

import jax
import jax.numpy as jnp
from jax import lax
from jax.experimental import pallas as pl
from jax.experimental.pallas import tpu as pltpu
from jax.experimental.pallas import tpu_sc as plsc

_NC = 2
_NS = 16
_NW = _NC * _NS
_FEAT = 16
_BLK = 128
_K = 4


_RING = 3


@functools.lru_cache(maxsize=None)
def _make_segment_kernel(n_nodes: int, e_pad: int, chunks: int,
                         acc_rows: int):
    wb_rows = acc_rows // _NS
    ch = _K * _BLK
    edges_per_tile = chunks * ch
    assert (chunks - 1) % _RING == 0

    mesh = plsc.VectorSubcoreMesh(
        core_axis_name="c", subcore_axis_name="s",
        num_cores=_NC, num_subcores=_NS)

    @functools.partial(
        pl.kernel,
        out_type=jax.ShapeDtypeStruct((2 * acc_rows, _FEAT), jnp.float32),
        mesh=mesh,
        scratch_types=[
            pltpu.VMEM((_RING, _K * _BLK), jnp.int32),
            pltpu.VMEM((_RING, _K * _BLK), jnp.int32),
            pltpu.VMEM((_RING, _K * _BLK, _FEAT), jnp.float32),
            pltpu.VMEM_SHARED((acc_rows, _FEAT), jnp.float32),
            pltpu.SemaphoreType.DMA,
            pltpu.SemaphoreType.DMA,
            pltpu.SemaphoreType.DMA,
        ],
        compiler_params=pltpu.CompilerParams(use_tc_tiling_on_sc=False),
    )
    def seg_kernel(h_hbm, src_hbm, dst_hbm, zeros_hbm, out_hbm,
                   src_v, dst_v, rows_v, acc, sem_i, sem_g, sem_s):
        c = lax.axis_index("c")
        s = lax.axis_index("s")
        wid = c * _NS + s

        pltpu.sync_copy(zeros_hbm, acc.at[pl.ds(s * wb_rows, wb_rows)])
        plsc.subcore_barrier()

        base = wid * edges_per_tile
        max_e0 = e_pad - ch

        def prefetch_idx(chunk, buf):
            e0 = jnp.minimum(base + chunk * ch, max_e0)
            pltpu.async_copy(src_hbm.at[pl.ds(e0, ch)], src_v.at[buf], sem_i)
            pltpu.async_copy(dst_hbm.at[pl.ds(e0, ch)], dst_v.at[buf], sem_i)

        def wait_idx(buf):
            pltpu.make_async_copy(
                src_hbm.at[pl.ds(0, ch)], src_v.at[buf], sem_i).wait()
            pltpu.make_async_copy(
                src_hbm.at[pl.ds(0, ch)], dst_v.at[buf], sem_i).wait()

        def issue_gathers(buf):
            pltpu.async_copy(h_hbm.at[src_v.at[buf]], rows_v.at[buf], sem_g)

        def drain_rows(buf, sem):
            pltpu.make_async_copy(h_hbm.at[pl.ds(0, ch)],
                                  rows_v.at[buf], sem).wait()

        def issue_scatters(buf):
            pltpu.async_copy(rows_v.at[buf], acc.at[dst_v.at[buf]],
                             sem_s, add=True)

        prefetch_idx(0, 0)
        prefetch_idx(1, 1)
        wait_idx(0)
        issue_gathers(0)

        groups = (chunks - 1) // _RING

        @pl.loop(0, groups)
        def _(g):
            for b in range(_RING):
                i = g * _RING + b
                if b < 2:
                    @pl.when(g > 0)
                    def _():
                        drain_rows(b, sem_s)
                else:
                    drain_rows(b, sem_s)
                prefetch_idx(i + 2, (b + 2) % _RING)
                wait_idx((b + 1) % _RING)
                issue_gathers((b + 1) % _RING)
                drain_rows(b, sem_g)
                issue_scatters(b)

        last = chunks - 1
        drain_rows(0, sem_g)
        issue_scatters(0)
        for b in ((last - 2) % _RING, (last - 1) % _RING, 0):
            drain_rows(b, sem_s)
        wait_idx(1)

        plsc.subcore_barrier()
        out_base = c * acc_rows + s * wb_rows
        pltpu.sync_copy(acc.at[pl.ds(s * wb_rows, wb_rows)],
                        out_hbm.at[pl.ds(out_base, wb_rows)])

    return seg_kernel


def kernel(x, edge_index, batch):
    n = x.shape[0]
    e = edge_index.shape[1]
    num_it = 3

    edges_per_sweep = _NW * _K * _BLK
    chunks = -(-e // edges_per_sweep)
    while (chunks - 1) % _RING:
        chunks += 1
    e_pad = chunks * edges_per_sweep
    acc_rows = ((n + 1 + 127) // 128) * 128
    wb_rows = acc_rows // _NS

    src = edge_index[0]
    dst = edge_index[1]
    pad = e_pad - e
    src_p = jnp.concatenate([src, jnp.zeros((pad,), jnp.int32)])
    dst_p = jnp.concatenate([dst, jnp.full((pad,), n, jnp.int32)])
    zeros = jnp.zeros((wb_rows, _FEAT), jnp.float32)

    h = jnp.concatenate(
        [
            x[:, :8],
            x[:, 9:10],
            x[:, 11:12],
            jnp.zeros((n, 5), jnp.float32),
            jnp.ones((n, 1), jnp.float32),
        ],
        axis=1,
    )

    seg = _make_segment_kernel(n, src_p.shape[0], chunks, acc_rows)

    snaps = [h]
    for _ in range(num_it):
        p = seg(h, src_p, dst_p, zeros)
        agg = p[:acc_rows][:n] + p[acc_rows:][:n]
        deg = jnp.maximum(agg[:, 15:16], 1.0)
        h = 0.5 * (h + agg / deg)
        h = h.at[:, 15].set(1.0)
        snaps.append(h)

    cont = jnp.concatenate([s[:, :8] for s in snaps], axis=1)
    cat1 = jnp.stack([s[:, 8] for s in snaps], axis=1)
    cat2 = jnp.stack([s[:, 9] for s in snaps], axis=1)
    return jnp.concatenate([cont, cat1, cat2], axis=1)


# speedup vs baseline: 188.6606x; 1.1167x over previous
import functools

import jax
import jax.numpy as jnp
from jax import lax
from jax.experimental import pallas as pl
from jax.experimental.pallas import tpu as pltpu
from jax.experimental.pallas import tpu_sc as plsc

_NC = 2
_NS = 16
_NW = _NC * _NS
_FEAT = 16
_BLK = 128
_K = 4


_RING = 3


@functools.lru_cache(maxsize=None)
def _make_segment_kernel(n_nodes: int, e_pad: int, chunks: int,
                         acc_rows: int):
    wb_rows = acc_rows // _NS
    ch = _K * _BLK
    edges_per_tile = chunks * ch
    assert (chunks - 1) % _RING == 0

    mesh = plsc.VectorSubcoreMesh(
        core_axis_name="c", subcore_axis_name="s",
        num_cores=_NC, num_subcores=_NS)

    @functools.partial(
        pl.kernel,
        out_type=jax.ShapeDtypeStruct((2 * acc_rows, _FEAT), jnp.float32),
        mesh=mesh,
        scratch_types=[
            pltpu.VMEM((_RING, _K * _BLK), jnp.int32),
            pltpu.VMEM((_RING, _K * _BLK), jnp.int32),
            pltpu.VMEM((_RING, _K * _BLK, _FEAT), jnp.float32),
            pltpu.VMEM_SHARED((acc_rows, _FEAT), jnp.float32),
            pltpu.SemaphoreType.DMA,
            pltpu.SemaphoreType.DMA,
            pltpu.SemaphoreType.DMA,
        ],
        compiler_params=pltpu.CompilerParams(use_tc_tiling_on_sc=False),
    )
    def seg_kernel(h_hbm, src_hbm, dst_hbm, zeros_hbm, out_hbm,
                   src_v, dst_v, rows_v, acc, sem_i, sem_g, sem_s):
        c = lax.axis_index("c")
        s = lax.axis_index("s")
        wid = c * _NS + s

        pltpu.sync_copy(zeros_hbm, acc.at[pl.ds(s * wb_rows, wb_rows)])
        plsc.subcore_barrier()

        base = wid * edges_per_tile
        max_e0 = e_pad - ch

        def prefetch_idx(chunk, buf):
            e0 = jnp.minimum(base + chunk * ch, max_e0)
            pltpu.async_copy(src_hbm.at[pl.ds(e0, ch)], src_v.at[buf], sem_i)
            pltpu.async_copy(dst_hbm.at[pl.ds(e0, ch)], dst_v.at[buf], sem_i)

        def wait_idx(buf):
            pltpu.make_async_copy(
                src_hbm.at[pl.ds(0, ch)], src_v.at[buf], sem_i).wait()
            pltpu.make_async_copy(
                src_hbm.at[pl.ds(0, ch)], dst_v.at[buf], sem_i).wait()

        def issue_gathers(buf):
            pltpu.async_copy(h_hbm.at[src_v.at[buf]], rows_v.at[buf], sem_g)

        def drain_rows(buf, sem):
            pltpu.make_async_copy(h_hbm.at[pl.ds(0, ch)],
                                  rows_v.at[buf], sem).wait()

        def issue_scatters(buf):
            pltpu.async_copy(rows_v.at[buf], acc.at[dst_v.at[buf]],
                             sem_s, add=True)

        prefetch_idx(0, 0)
        prefetch_idx(1, 1)
        wait_idx(0)
        issue_gathers(0)

        groups = (chunks - 1) // _RING

        @pl.loop(0, groups)
        def _(g):
            for b in range(_RING):
                i = g * _RING + b
                if b < 2:
                    @pl.when(g > 0)
                    def _():
                        drain_rows(b, sem_s)
                else:
                    drain_rows(b, sem_s)
                prefetch_idx(i + 2, (b + 2) % _RING)
                wait_idx((b + 1) % _RING)
                issue_gathers((b + 1) % _RING)
                drain_rows(b, sem_g)
                issue_scatters(b)

        last = chunks - 1
        drain_rows(0, sem_g)
        issue_scatters(0)
        for b in ((last - 2) % _RING, (last - 1) % _RING, 0):
            drain_rows(b, sem_s)
        wait_idx(1)

        plsc.subcore_barrier()
        out_base = c * acc_rows + s * wb_rows
        pltpu.sync_copy(acc.at[pl.ds(s * wb_rows, wb_rows)],
                        out_hbm.at[pl.ds(out_base, wb_rows)])

    return seg_kernel


def kernel(x, edge_index, batch):
    n = x.shape[0]
    e = edge_index.shape[1]
    num_it = 3

    edges_per_sweep = _NW * _K * _BLK
    chunks = -(-e // edges_per_sweep)
    while (chunks - 1) % _RING:
        chunks += 1
    e_pad = chunks * edges_per_sweep
    acc_rows = ((n + 1 + 127) // 128) * 128
    wb_rows = acc_rows // _NS

    src = edge_index[0]
    dst = edge_index[1]
    pad = e_pad - e
    src_p = jnp.concatenate([src, jnp.zeros((pad,), jnp.int32)])
    dst_p = jnp.concatenate([dst, jnp.full((pad,), n, jnp.int32)])
    zeros = jnp.zeros((wb_rows, _FEAT), jnp.float32)

    h = jnp.concatenate(
        [
            x[:, :8],
            x[:, 9:10],
            x[:, 11:12],
            jnp.zeros((n, 5), jnp.float32),
            jnp.ones((n, 1), jnp.float32),
        ],
        axis=1,
    )

    seg = _make_segment_kernel(n, src_p.shape[0], chunks, acc_rows)

    is_ones_col = (jnp.arange(_FEAT) == _FEAT - 1)[None, :]
    snaps = [h]
    for _ in range(num_it):
        p = seg(h, src_p, dst_p, zeros)
        agg = p[:acc_rows][:n] + p[acc_rows:][:n]
        deg = jnp.maximum(agg[:, 15:16], 1.0)
        h = jnp.where(is_ones_col, 1.0, 0.5 * (h + agg / deg))
        snaps.append(h)

    cont = jnp.concatenate([s[:, :8] for s in snaps], axis=1)
    cat1 = jnp.stack([s[:, 8] for s in snaps], axis=1)
    cat2 = jnp.stack([s[:, 9] for s in snaps], axis=1)
    return jnp.concatenate([cont, cat1, cat2], axis=1)


# speedup vs baseline: 195.2646x; 1.0350x over previous
import functools

import jax
import jax.numpy as jnp
from jax import lax
from jax.experimental import pallas as pl
from jax.experimental.pallas import tpu as pltpu
from jax.experimental.pallas import tpu_sc as plsc

_NC = 2
_NS = 16
_NW = _NC * _NS
_FEAT = 16
_CH = 320
_RING = 3
_UCH = 184
_NUM_IT = 3


@functools.lru_cache(maxsize=None)
def _make_wl_kernel(e_pad: int, chunks: int, acc_rows: int):
    wb_rows = acc_rows // _NS
    urows = acc_rows // _NW
    uch = urows // _UCH
    edges_per_tile = chunks * _CH
    assert (chunks - 1) % _RING == 0
    assert urows % _UCH == 0

    mesh = plsc.VectorSubcoreMesh(
        core_axis_name="c", subcore_axis_name="s",
        num_cores=_NC, num_subcores=_NS)

    table_t = jax.ShapeDtypeStruct((acc_rows, _FEAT), jnp.float32)

    @functools.partial(
        pl.kernel,
        out_type=(table_t, table_t, table_t,
                  jax.ShapeDtypeStruct((_NC * acc_rows, _FEAT), jnp.float32)),
        mesh=mesh,
        scratch_types=[
            pltpu.VMEM((_RING, _CH), jnp.int32),
            pltpu.VMEM((_RING, _CH), jnp.int32),
            pltpu.VMEM((_RING, _CH, _FEAT), jnp.float32),
            pltpu.VMEM((_UCH, _FEAT), jnp.float32),
            pltpu.VMEM((_UCH, _FEAT), jnp.float32),
            pltpu.VMEM((_UCH, _FEAT), jnp.float32),
            pltpu.VMEM_SHARED((acc_rows, _FEAT), jnp.float32),
            pltpu.SemaphoreType.DMA,
            pltpu.SemaphoreType.DMA,
            pltpu.SemaphoreType.DMA,
            pltpu.SemaphoreType.REGULAR,
        ],
        compiler_params=pltpu.CompilerParams(
            use_tc_tiling_on_sc=False, needs_layout_passes=False),
    )
    def wl_kernel(h0_hbm, src_hbm, dst_hbm, zeros_hbm,
                  out1, out2, out3, p_hbm,
                  src_v, dst_v, rows_v, po_v, own_v, h_v, acc,
                  sem_i, sem_g, sem_s, sem_x):
        c = lax.axis_index("c")
        s = lax.axis_index("s")
        wid = c * _NS + s
        base = wid * edges_per_tile
        max_e0 = e_pad - _CH
        lane = lax.iota(jnp.int32, 16)

        def cross_core_barrier():
            plsc.subcore_barrier()

            @pl.when(s == 0)
            def _():
                pl.semaphore_signal(sem_x, 1, core_index=1 - c)
                pl.semaphore_wait(sem_x, 1)

            plsc.subcore_barrier()

        def prefetch_idx(chunk, buf):
            e0 = jnp.minimum(base + chunk * _CH, max_e0)
            pltpu.async_copy(src_hbm.at[pl.ds(e0, _CH)], src_v.at[buf], sem_i)
            pltpu.async_copy(dst_hbm.at[pl.ds(e0, _CH)], dst_v.at[buf], sem_i)

        def wait_idx(buf):
            pltpu.make_async_copy(
                src_hbm.at[pl.ds(0, _CH)], src_v.at[buf], sem_i).wait()
            pltpu.make_async_copy(
                src_hbm.at[pl.ds(0, _CH)], dst_v.at[buf], sem_i).wait()

        def drain_rows(buf, sem):
            pltpu.make_async_copy(h0_hbm.at[pl.ds(0, _CH)],
                                  rows_v.at[buf], sem).wait()

        def scatter_phase(table):
            pltpu.sync_copy(zeros_hbm, acc.at[pl.ds(s * wb_rows, wb_rows)])
            plsc.subcore_barrier()

            def issue_gathers(buf):
                pltpu.async_copy(table.at[src_v.at[buf]], rows_v.at[buf],
                                 sem_g)

            def issue_scatters(buf):
                pltpu.async_copy(rows_v.at[buf], acc.at[dst_v.at[buf]],
                                 sem_s, add=True)

            prefetch_idx(0, 0)
            prefetch_idx(1, 1)
            wait_idx(0)
            issue_gathers(0)

            groups = (chunks - 1) // _RING

            @pl.loop(0, groups)
            def _(g):
                for b in range(_RING):
                    i = g * _RING + b
                    if b < 2:
                        @pl.when(g > 0)
                        def _():
                            drain_rows(b, sem_s)
                    else:
                        drain_rows(b, sem_s)
                    prefetch_idx(i + 2, (b + 2) % _RING)
                    wait_idx((b + 1) % _RING)
                    issue_gathers((b + 1) % _RING)
                    drain_rows(b, sem_g)
                    issue_scatters(b)

            drain_rows(0, sem_g)
            issue_scatters(0)
            for b in range(_RING):
                drain_rows(b, sem_s)
            wait_idx(1)
            plsc.subcore_barrier()

        def update_phase(table_prev, table_out):
            pltpu.sync_copy(
                acc.at[pl.ds(s * wb_rows, wb_rows)],
                p_hbm.at[pl.ds(c * acc_rows + s * wb_rows, wb_rows)])
            cross_core_barrier()

            @pl.loop(0, uch)
            def _(u):
                r0 = wid * urows + u * _UCH
                pltpu.sync_copy(
                    p_hbm.at[pl.ds((1 - c) * acc_rows + r0, _UCH)], po_v)
                pltpu.sync_copy(acc.at[pl.ds(r0, _UCH)], own_v)
                pltpu.sync_copy(table_prev.at[pl.ds(r0, _UCH)], h_v)

                @pl.loop(0, _UCH)
                def _(r):
                    agg = own_v[r] + po_v[r]
                    d = jnp.maximum(jnp.max(jnp.where(lane == 15, agg, -1.0)),
                                    1.0)
                    hn = 0.5 * (h_v[r] + agg / d)
                    h_v[r] = jnp.where(lane == 15, 1.0, hn)

                pltpu.sync_copy(h_v, table_out.at[pl.ds(r0, _UCH)])

            cross_core_barrier()

        tables = [h0_hbm, out1, out2, out3]
        for t in range(_NUM_IT):
            scatter_phase(tables[t])
            update_phase(tables[t], tables[t + 1])

    return wl_kernel


def kernel(x, edge_index, batch):
    n = x.shape[0]
    e = edge_index.shape[1]

    edges_per_sweep = _NW * _CH
    chunks = -(-e // edges_per_sweep)
    while (chunks - 1) % _RING:
        chunks += 1
    e_pad = chunks * edges_per_sweep
    acc_rows = -(-(n + 1) // (_NW * _UCH)) * (_NW * _UCH)
    wb_rows = acc_rows // _NS

    src = edge_index[0]
    dst = edge_index[1]
    pad = e_pad - e
    if pad:
        src = jnp.concatenate([src, jnp.zeros((pad,), jnp.int32)])
        dst = jnp.concatenate([dst, jnp.full((pad,), n, jnp.int32)])
    zeros = jnp.zeros((wb_rows, _FEAT), jnp.float32)

    h0 = jnp.concatenate(
        [
            x[:, :8],
            x[:, 9:10],
            x[:, 11:12],
            jnp.zeros((n, 5), jnp.float32),
            jnp.ones((n, 1), jnp.float32),
        ],
        axis=1,
    )
    h0 = jnp.concatenate(
        [h0, jnp.zeros((acc_rows - n, _FEAT), jnp.float32)], axis=0)

    wl = _make_wl_kernel(e_pad, chunks, acc_rows)
    h1, h2, h3, _ = wl(h0, src, dst, zeros)

    snaps = [h0[:n], h1[:n], h2[:n], h3[:n]]
    cont = jnp.concatenate([s[:, :8] for s in snaps], axis=1)
    cat1 = jnp.stack([s[:, 8] for s in snaps], axis=1)
    cat2 = jnp.stack([s[:, 9] for s in snaps], axis=1)
    return jnp.concatenate([cont, cat1, cat2], axis=1)
